# baseline (speedup 1.0000x reference)
import functools

import jax
import jax.numpy as jnp
from jax import lax
from jax.experimental import pallas as pl
from jax.experimental.pallas import tpu as pltpu
from jax.experimental.pallas import tpu_sc as plsc

N = 10000
E = 320000
D = 128
RB = 8
AVG_NEIGH = 32.0

NB = 2000

NC = 2
NS = 16
NW = NC * NS
CH = 128

STRIPES = ((2500, 2000),)

N_PAD = 10240
_RPT = N_PAD // NS
_ZR = 32

_sc_mesh = plsc.VectorSubcoreMesh(core_axis_name="c", subcore_axis_name="s")


def _node_kernel(nf_ref, wsc_ref, w1a_ref, w1b_ref, wup_ref, au_ref, b_ref):
    nf = nf_ref[...]
    ns = jnp.dot(nf, wsc_ref[...], preferred_element_type=jnp.float32)
    a = jnp.dot(ns, w1a_ref[...], preferred_element_type=jnp.float32)
    u = jnp.dot(nf, wup_ref[...], preferred_element_type=jnp.float32)
    ai = lax.bitcast_convert_type(a.astype(jnp.bfloat16).astype(jnp.float32),
                                  jnp.int32)
    ui = lax.bitcast_convert_type(u.astype(jnp.bfloat16).astype(jnp.float32),
                                  jnp.int32)
    au_ref[...] = lax.bitcast_convert_type(
        ai | lax.shift_right_logical(ui, 16), jnp.float32)
    b_ref[...] = jnp.dot(ns, w1b_ref[...], preferred_element_type=jnp.float32)


def _node_precompute(node_feats, W_scalar, W1a, W1b, W_up):
    grid = (N // NB,)
    blk = pl.BlockSpec((NB, D), lambda i: (i, 0))
    wblk = pl.BlockSpec((D, D), lambda i: (0, 0))
    return pl.pallas_call(
        _node_kernel,
        grid=grid,
        in_specs=[blk, wblk, wblk, wblk, wblk],
        out_specs=[blk, blk],
        out_shape=[jax.ShapeDtypeStruct((N, D), jnp.float32),
                   jax.ShapeDtypeStruct((N, D), jnp.float32)],
    )(node_feats, W_scalar, W1a, W1b, W_up)


def _make_gather(nchunk):
    base = nchunk // NW
    extra = nchunk - base * NW
    ne = nchunk * CH

    @functools.partial(
        pl.kernel,
        out_type=[jax.ShapeDtypeStruct((ne, D), jnp.float32),
                  jax.ShapeDtypeStruct((ne, D), jnp.float32)],
        mesh=_sc_mesh,
        scratch_types=[
            pltpu.VMEM((2, CH), jnp.int32),
            pltpu.VMEM((2, CH), jnp.int32),
            pltpu.VMEM((CH, D), jnp.float32),
            pltpu.VMEM((CH, D), jnp.float32),
            pltpu.VMEM((CH, D), jnp.float32),
            pltpu.VMEM((CH, D), jnp.float32),
            pltpu.SemaphoreType.DMA,
            pltpu.SemaphoreType.DMA,
            pltpu.SemaphoreType.DMA,
            pltpu.SemaphoreType.DMA,
            pltpu.SemaphoreType.DMA,
            pltpu.SemaphoreType.DMA,
        ],
    )
    def gather(au_hbm, b_hbm, ei_hbm, gp_hbm, gr_hbm,
               idx0, idx1, au0, au1, b0, b1, si0, si1, sg0, sg1, sw0, sw1):
        wid = lax.axis_index("s") * NC + lax.axis_index("c")

        def start_idx(j, idxb, sem):
            pltpu.async_copy(ei_hbm.at[wid + j * NW], idxb, sem)

        def wait_idx(idxb, sem):
            pltpu.make_async_copy(ei_hbm.at[0], idxb, sem).wait()

        def start_gather(idxb, aub, bb, sem):
            pltpu.async_copy(au_hbm.at[idxb.at[0]], aub, sem)
            pltpu.async_copy(b_hbm.at[idxb.at[1]], bb, sem)

        def wait_gather(idxb, aub, bb, sem):
            pltpu.make_async_copy(au_hbm.at[idxb.at[0]], aub, sem).wait()
            pltpu.make_async_copy(b_hbm.at[idxb.at[1]], bb, sem).wait()

        def start_write(j, aub, bb, sem):
            cid = wid + j * NW
            pltpu.async_copy(aub, gp_hbm.at[pl.ds(cid * CH, CH)], sem)
            pltpu.async_copy(bb, gr_hbm.at[pl.ds(cid * CH, CH)], sem)

        def wait_write(aub, bb, sem):
            pltpu.make_async_copy(aub, gp_hbm.at[pl.ds(0, CH)], sem).wait()
            pltpu.make_async_copy(bb, gr_hbm.at[pl.ds(0, CH)], sem).wait()

        start_idx(0, idx0, si0)
        start_idx(1, idx1, si1)

        def body(jj, carry):
            wait_idx(idx0, si0)

            @pl.when(jj > 0)
            def _():
                wait_write(au0, b0, sw0)

            start_gather(idx0, au0, b0, sg0)
            wait_idx(idx1, si1)

            @pl.when(jj > 0)
            def _():
                wait_write(au1, b1, sw1)

            start_gather(idx1, au1, b1, sg1)
            wait_gather(idx0, au0, b0, sg0)

            @pl.when(jj < base // 2 - 1)
            def _():
                start_idx(2 * jj + 2, idx0, si0)

            start_write(2 * jj, au0, b0, sw0)
            wait_gather(idx1, au1, b1, sg1)

            @pl.when(jj < base // 2 - 1)
            def _():
                start_idx(2 * jj + 3, idx1, si1)

            start_write(2 * jj + 1, au1, b1, sw1)
            return carry

        lax.fori_loop(0, base // 2, body, 0)
        wait_write(au0, b0, sw0)
        wait_write(au1, b1, sw1)

        if extra:
            @pl.when(wid < extra)
            def _():
                start_idx(base, idx0, si0)
                wait_idx(idx0, si0)
                start_gather(idx0, au0, b0, sg0)
                wait_gather(idx0, au0, b0, sg0)
                start_write(base, au0, b0, sw0)
                wait_write(au0, b0, sw0)

    return gather


def _make_scatter(nchunk):
    base = nchunk // NW
    extra = nchunk - base * NW

    @functools.partial(
        pl.kernel,
        out_type=jax.ShapeDtypeStruct((NC, N_PAD, D), jnp.float32),
        mesh=_sc_mesh,
        scratch_types=[
            pltpu.VMEM((CH,), jnp.int32),
            pltpu.VMEM((CH,), jnp.int32),
            pltpu.VMEM((CH, D), jnp.float32),
            pltpu.VMEM((CH, D), jnp.float32),
            pltpu.VMEM((_ZR, D), jnp.float32),
            pltpu.VMEM_SHARED((N_PAD, D), jnp.float32),
            pltpu.SemaphoreType.DMA,
            pltpu.SemaphoreType.DMA,
        ],
    )
    def scatter(mji_hbm, ri_hbm, out_hbm, idx0, idx1, rows0, rows1, zbuf,
                acc, sl0, sl1):
        c = lax.axis_index("c")
        s = lax.axis_index("s")
        wid = s * NC + c

        def zrow(i, carry):
            for k in range(D // 16):
                zbuf[i, pl.ds(k * 16, 16)] = jnp.zeros((16,), jnp.float32)
            return carry

        lax.fori_loop(0, _ZR, zrow, 0)
        for t in range(_RPT // _ZR):
            pltpu.sync_copy(zbuf, acc.at[pl.ds(s * _RPT + t * _ZR, _ZR)])
        plsc.subcore_barrier()

        def start_load(j, idxb, rowsb, sem):
            cid = wid + j * NW
            pltpu.async_copy(ri_hbm.at[cid], idxb, sem)
            pltpu.async_copy(mji_hbm.at[pl.ds(cid * CH, CH)], rowsb, sem)

        def wait_load(idxb, rowsb, sem):
            pltpu.make_async_copy(ri_hbm.at[0], idxb, sem).wait()
            pltpu.make_async_copy(mji_hbm.at[pl.ds(0, CH)], rowsb, sem).wait()

        start_load(0, idx0, rows0, sl0)
        start_load(1, idx1, rows1, sl1)

        def body(jj, carry):
            wait_load(idx0, rows0, sl0)
            pltpu.sync_copy(rows0, acc.at[idx0], add=True)

            @pl.when(jj < base // 2 - 1)
            def _():
                start_load(2 * jj + 2, idx0, rows0, sl0)

            wait_load(idx1, rows1, sl1)
            pltpu.sync_copy(rows1, acc.at[idx1], add=True)

            @pl.when(jj < base // 2 - 1)
            def _():
                start_load(2 * jj + 3, idx1, rows1, sl1)

            return carry

        lax.fori_loop(0, base // 2, body, 0)

        if extra:
            @pl.when(wid < extra)
            def _():
                start_load(base, idx0, rows0, sl0)
                wait_load(idx0, rows0, sl0)
                pltpu.sync_copy(rows0, acc.at[idx0], add=True)

        plsc.subcore_barrier()
        sl = pl.ds(s * _RPT, _RPT)
        pltpu.sync_copy(acc.at[sl], out_hbm.at[c, sl])

    return scatter


_gathers = tuple(_make_gather(nc) for nc, _ in STRIPES)
_scatters = tuple(_make_scatter(nc) for nc, _ in STRIPES)


def _mlp_kernel(gp_ref, gr_ref, ef_ref, ea_ref,
                w1c_ref, w2_ref, b2_ref, w3_ref, o_ref):
    xi = lax.bitcast_convert_type(gp_ref[...], jnp.int32)
    gs = lax.bitcast_convert_type(xi & jnp.int32(-65536), jnp.float32)
    u = lax.bitcast_convert_type(lax.shift_left(xi, 16), jnp.float32)
    pre = gs + gr_ref[...]
    pre = pre + jnp.dot(ef_ref[...], w1c_ref[...],
                        preferred_element_type=jnp.float32)
    h = pre * jax.nn.sigmoid(pre)
    pre2 = jnp.dot(h.astype(jnp.bfloat16), w2_ref[...],
                   preferred_element_type=jnp.float32) + b2_ref[...]
    h2 = pre2 * jax.nn.sigmoid(pre2)
    t = jnp.dot(h2.astype(jnp.bfloat16), w3_ref[...],
                preferred_element_type=jnp.float32)
    o_ref[...] = u * ea_ref[...] * t


def _edge_mlp(gp, gr, ef_aug, ea, W1c_aug, W2, b2, W3, eb):
    ne = gp.shape[0]
    grid = (ne // eb,)
    eblk = pl.BlockSpec((eb, D), lambda i: (i, 0))
    wblk = pl.BlockSpec((D, D), lambda i: (0, 0))
    rblk = pl.BlockSpec((1, D), lambda i: (0, 0))
    return pl.pallas_call(
        _mlp_kernel,
        grid=grid,
        in_specs=[
            eblk, eblk,
            pl.BlockSpec((eb, RB + 8), lambda i: (i, 0)),
            pl.BlockSpec((eb, 1), lambda i: (i, 0)),
            pl.BlockSpec((RB + 8, D), lambda i: (0, 0)),
            wblk, rblk, wblk,
        ],
        out_specs=eblk,
        out_shape=jax.ShapeDtypeStruct((ne, D), jnp.float32),
    )(gp, gr, ef_aug, ea, W1c_aug, W2, b2, W3)


def _final_kernel(p0_ref, wout_ref, o_ref):
    m = p0_ref[0] + p0_ref[1]
    o_ref[...] = jnp.dot(m, wout_ref[...],
                         preferred_element_type=jnp.float32) * (1.0 / AVG_NEIGH)


def _final(p0, W_out):
    grid = (N // NB,)
    blk = pl.BlockSpec((NB, D), lambda i: (i, 0))
    pblk = pl.BlockSpec((NC, NB, D), lambda i: (0, i, 0))
    return pl.pallas_call(
        _final_kernel,
        grid=grid,
        in_specs=[pblk, pl.BlockSpec((D, D), lambda i: (0, 0))],
        out_specs=blk,
        out_shape=jax.ShapeDtypeStruct((N, D), jnp.float32),
    )(p0, W_out)


def kernel(node_feats, edge_attrs, edge_feats, lengths, W_scalar, W_up,
           W1, b1, W2, b2, W3, W_out, edge_index):
    W1a = W1[:D]
    W1b = W1[D:2 * D]
    W1c_aug = jnp.concatenate(
        [W1[2 * D:], b1[None, :], jnp.zeros((16 - RB - 2, D), jnp.float32)],
        axis=0)
    ef_aug = jnp.concatenate(
        [edge_feats, lengths, jnp.ones((E, 1), jnp.float32),
         jnp.zeros((E, 16 - RB - 2), jnp.float32)], axis=1)

    au, b = _node_precompute(node_feats, W_scalar, W1a, W1b, W_up)

    nchunk = E // CH
    ei3 = edge_index.reshape(2, nchunk, CH).transpose(1, 0, 2)
    ri = edge_index[1].reshape(nchunk, CH)

    partials = []
    off_c = 0
    for (nc_s, eb_s), g_call, s_call in zip(STRIPES, _gathers, _scatters):
        ne_s = nc_s * CH
        e0 = off_c * CH
        gp, gr = g_call(au, b, ei3[off_c:off_c + nc_s])
        mji = _edge_mlp(gp, gr,
                        ef_aug[e0:e0 + ne_s],
                        edge_attrs[e0:e0 + ne_s],
                        W1c_aug, W2.astype(jnp.bfloat16), b2[None, :],
                        W3.astype(jnp.bfloat16), eb_s)
        partials.append(s_call(mji, ri[off_c:off_c + nc_s]))
        off_c += nc_s

    out = _final(partials[0], W_out)
    return out.reshape(N, D, 1)


# speedup vs baseline: 1.0663x; 1.0663x over previous
import functools

import jax
import jax.numpy as jnp
from jax import lax
from jax.experimental import pallas as pl
from jax.experimental.pallas import tpu as pltpu
from jax.experimental.pallas import tpu_sc as plsc

N = 10000
E = 320000
D = 128
RB = 8
AVG_NEIGH = 32.0

NB = 2000

NC = 2
NS = 16
NW = NC * NS
CH = 128

STRIPES = ((2500, 4000),)

N_PAD = 10240
_RPT = N_PAD // NS
_ZR = 32

_sc_mesh = plsc.VectorSubcoreMesh(core_axis_name="c", subcore_axis_name="s")


def _node_kernel(nf_ref, wsc_ref, w1a_ref, w1b_ref, wup_ref, au_ref, b_ref):
    nf = nf_ref[...]
    ns = jnp.dot(nf, wsc_ref[...], preferred_element_type=jnp.float32)
    a = jnp.dot(ns, w1a_ref[...], preferred_element_type=jnp.float32)
    u = jnp.dot(nf, wup_ref[...], preferred_element_type=jnp.float32)
    ai = lax.bitcast_convert_type(a.astype(jnp.bfloat16).astype(jnp.float32),
                                  jnp.int32)
    ui = lax.bitcast_convert_type(u.astype(jnp.bfloat16).astype(jnp.float32),
                                  jnp.int32)
    au_ref[...] = lax.bitcast_convert_type(
        ai | lax.shift_right_logical(ui, 16), jnp.float32)
    b_ref[...] = jnp.dot(ns, w1b_ref[...], preferred_element_type=jnp.float32)


def _node_precompute(node_feats, W_scalar, W1a, W1b, W_up):
    grid = (N // NB,)
    blk = pl.BlockSpec((NB, D), lambda i: (i, 0))
    wblk = pl.BlockSpec((D, D), lambda i: (0, 0))
    return pl.pallas_call(
        _node_kernel,
        grid=grid,
        in_specs=[blk, wblk, wblk, wblk, wblk],
        out_specs=[blk, blk],
        out_shape=[jax.ShapeDtypeStruct((N, D), jnp.float32),
                   jax.ShapeDtypeStruct((N, D), jnp.float32)],
    )(node_feats, W_scalar, W1a, W1b, W_up)


def _make_gather(nchunk):
    base = nchunk // NW
    extra = nchunk - base * NW
    ne = nchunk * CH

    @functools.partial(
        pl.kernel,
        out_type=[jax.ShapeDtypeStruct((ne, D), jnp.float32),
                  jax.ShapeDtypeStruct((ne, D), jnp.float32)],
        mesh=_sc_mesh,
        scratch_types=[
            pltpu.VMEM((2, CH), jnp.int32),
            pltpu.VMEM((2, CH), jnp.int32),
            pltpu.VMEM((CH, D), jnp.float32),
            pltpu.VMEM((CH, D), jnp.float32),
            pltpu.VMEM((CH, D), jnp.float32),
            pltpu.VMEM((CH, D), jnp.float32),
            pltpu.SemaphoreType.DMA,
            pltpu.SemaphoreType.DMA,
            pltpu.SemaphoreType.DMA,
            pltpu.SemaphoreType.DMA,
            pltpu.SemaphoreType.DMA,
            pltpu.SemaphoreType.DMA,
        ],
    )
    def gather(au_hbm, b_hbm, ei_hbm, gp_hbm, gr_hbm,
               idx0, idx1, au0, au1, b0, b1, si0, si1, sg0, sg1, sw0, sw1):
        wid = lax.axis_index("s") * NC + lax.axis_index("c")

        def start_idx(j, idxb, sem):
            pltpu.async_copy(ei_hbm.at[wid + j * NW], idxb, sem)

        def wait_idx(idxb, sem):
            pltpu.make_async_copy(ei_hbm.at[0], idxb, sem).wait()

        def start_gather(idxb, aub, bb, sem):
            pltpu.async_copy(au_hbm.at[idxb.at[0]], aub, sem)
            pltpu.async_copy(b_hbm.at[idxb.at[1]], bb, sem)

        def wait_gather(idxb, aub, bb, sem):
            pltpu.make_async_copy(au_hbm.at[idxb.at[0]], aub, sem).wait()
            pltpu.make_async_copy(b_hbm.at[idxb.at[1]], bb, sem).wait()

        def start_write(j, aub, bb, sem):
            cid = wid + j * NW
            pltpu.async_copy(aub, gp_hbm.at[pl.ds(cid * CH, CH)], sem)
            pltpu.async_copy(bb, gr_hbm.at[pl.ds(cid * CH, CH)], sem)

        def wait_write(aub, bb, sem):
            pltpu.make_async_copy(aub, gp_hbm.at[pl.ds(0, CH)], sem).wait()
            pltpu.make_async_copy(bb, gr_hbm.at[pl.ds(0, CH)], sem).wait()

        start_idx(0, idx0, si0)
        start_idx(1, idx1, si1)

        def body(jj, carry):
            wait_idx(idx0, si0)

            @pl.when(jj > 0)
            def _():
                wait_write(au0, b0, sw0)

            start_gather(idx0, au0, b0, sg0)
            wait_idx(idx1, si1)

            @pl.when(jj > 0)
            def _():
                wait_write(au1, b1, sw1)

            start_gather(idx1, au1, b1, sg1)
            wait_gather(idx0, au0, b0, sg0)

            @pl.when(jj < base // 2 - 1)
            def _():
                start_idx(2 * jj + 2, idx0, si0)

            start_write(2 * jj, au0, b0, sw0)
            wait_gather(idx1, au1, b1, sg1)

            @pl.when(jj < base // 2 - 1)
            def _():
                start_idx(2 * jj + 3, idx1, si1)

            start_write(2 * jj + 1, au1, b1, sw1)
            return carry

        lax.fori_loop(0, base // 2, body, 0)
        wait_write(au0, b0, sw0)
        wait_write(au1, b1, sw1)

        if extra:
            @pl.when(wid < extra)
            def _():
                start_idx(base, idx0, si0)
                wait_idx(idx0, si0)
                start_gather(idx0, au0, b0, sg0)
                wait_gather(idx0, au0, b0, sg0)
                start_write(base, au0, b0, sw0)
                wait_write(au0, b0, sw0)

    return gather


def _make_scatter(nchunk):
    base = nchunk // NW
    extra = nchunk - base * NW

    @functools.partial(
        pl.kernel,
        out_type=jax.ShapeDtypeStruct((NC, N_PAD, D), jnp.float32),
        mesh=_sc_mesh,
        scratch_types=[
            pltpu.VMEM((CH,), jnp.int32),
            pltpu.VMEM((CH,), jnp.int32),
            pltpu.VMEM((CH, D), jnp.float32),
            pltpu.VMEM((CH, D), jnp.float32),
            pltpu.VMEM((_ZR, D), jnp.float32),
            pltpu.VMEM_SHARED((N_PAD, D), jnp.float32),
            pltpu.SemaphoreType.DMA,
            pltpu.SemaphoreType.DMA,
        ],
    )
    def scatter(mji_hbm, ri_hbm, out_hbm, idx0, idx1, rows0, rows1, zbuf,
                acc, sl0, sl1):
        c = lax.axis_index("c")
        s = lax.axis_index("s")
        wid = s * NC + c

        def zrow(i, carry):
            for k in range(D // 16):
                zbuf[i, pl.ds(k * 16, 16)] = jnp.zeros((16,), jnp.float32)
            return carry

        lax.fori_loop(0, _ZR, zrow, 0)
        for t in range(_RPT // _ZR):
            pltpu.sync_copy(zbuf, acc.at[pl.ds(s * _RPT + t * _ZR, _ZR)])
        plsc.subcore_barrier()

        def start_load(j, idxb, rowsb, sem):
            cid = wid + j * NW
            pltpu.async_copy(ri_hbm.at[cid], idxb, sem)
            pltpu.async_copy(mji_hbm.at[pl.ds(cid * CH, CH)], rowsb, sem)

        def wait_load(idxb, rowsb, sem):
            pltpu.make_async_copy(ri_hbm.at[0], idxb, sem).wait()
            pltpu.make_async_copy(mji_hbm.at[pl.ds(0, CH)], rowsb, sem).wait()

        start_load(0, idx0, rows0, sl0)
        start_load(1, idx1, rows1, sl1)

        def body(jj, carry):
            wait_load(idx0, rows0, sl0)
            pltpu.sync_copy(rows0, acc.at[idx0], add=True)

            @pl.when(jj < base // 2 - 1)
            def _():
                start_load(2 * jj + 2, idx0, rows0, sl0)

            wait_load(idx1, rows1, sl1)
            pltpu.sync_copy(rows1, acc.at[idx1], add=True)

            @pl.when(jj < base // 2 - 1)
            def _():
                start_load(2 * jj + 3, idx1, rows1, sl1)

            return carry

        lax.fori_loop(0, base // 2, body, 0)

        if extra:
            @pl.when(wid < extra)
            def _():
                start_load(base, idx0, rows0, sl0)
                wait_load(idx0, rows0, sl0)
                pltpu.sync_copy(rows0, acc.at[idx0], add=True)

        plsc.subcore_barrier()
        sl = pl.ds(s * _RPT, _RPT)
        pltpu.sync_copy(acc.at[sl], out_hbm.at[c, sl])

    return scatter


_gathers = tuple(_make_gather(nc) for nc, _ in STRIPES)
_scatters = tuple(_make_scatter(nc) for nc, _ in STRIPES)


def _mlp_kernel(gp_ref, gr_ref, ef_ref, ea_ref,
                w1c_ref, w2_ref, b2_ref, w3_ref, o_ref):
    xi = lax.bitcast_convert_type(gp_ref[...], jnp.int32)
    gs = lax.bitcast_convert_type(xi & jnp.int32(-65536), jnp.float32)
    u = lax.bitcast_convert_type(lax.shift_left(xi, 16), jnp.float32)
    pre = gs + gr_ref[...]
    pre = pre + jnp.dot(ef_ref[...], w1c_ref[...],
                        preferred_element_type=jnp.float32)
    h = pre * jax.nn.sigmoid(pre)
    pre2 = jnp.dot(h, w2_ref[...], preferred_element_type=jnp.float32) + b2_ref[...]
    h2 = pre2 * jax.nn.sigmoid(pre2)
    t = jnp.dot(h2, w3_ref[...], preferred_element_type=jnp.float32)
    o_ref[...] = u * ea_ref[...] * t


def _edge_mlp(gp, gr, ef_aug, ea, W1c_aug, W2, b2, W3, eb):
    ne = gp.shape[0]
    grid = (ne // eb,)
    eblk = pl.BlockSpec((eb, D), lambda i: (i, 0))
    wblk = pl.BlockSpec((D, D), lambda i: (0, 0))
    rblk = pl.BlockSpec((1, D), lambda i: (0, 0))
    return pl.pallas_call(
        _mlp_kernel,
        grid=grid,
        in_specs=[
            eblk, eblk,
            pl.BlockSpec((eb, RB + 8), lambda i: (i, 0)),
            pl.BlockSpec((eb, 1), lambda i: (i, 0)),
            pl.BlockSpec((RB + 8, D), lambda i: (0, 0)),
            wblk, rblk, wblk,
        ],
        out_specs=eblk,
        out_shape=jax.ShapeDtypeStruct((ne, D), jnp.float32),
    )(gp, gr, ef_aug, ea, W1c_aug, W2, b2, W3)


def _final_kernel(p0_ref, wout_ref, o_ref):
    m = p0_ref[0] + p0_ref[1]
    o_ref[...] = jnp.dot(m, wout_ref[...],
                         preferred_element_type=jnp.float32) * (1.0 / AVG_NEIGH)


def _final(p0, W_out):
    grid = (N // NB,)
    blk = pl.BlockSpec((NB, D), lambda i: (i, 0))
    pblk = pl.BlockSpec((NC, NB, D), lambda i: (0, i, 0))
    return pl.pallas_call(
        _final_kernel,
        grid=grid,
        in_specs=[pblk, pl.BlockSpec((D, D), lambda i: (0, 0))],
        out_specs=blk,
        out_shape=jax.ShapeDtypeStruct((N, D), jnp.float32),
    )(p0, W_out)


def kernel(node_feats, edge_attrs, edge_feats, lengths, W_scalar, W_up,
           W1, b1, W2, b2, W3, W_out, edge_index):
    W1a = W1[:D]
    W1b = W1[D:2 * D]
    W1c_aug = jnp.concatenate(
        [W1[2 * D:], b1[None, :], jnp.zeros((16 - RB - 2, D), jnp.float32)],
        axis=0)
    ef_aug = jnp.concatenate(
        [edge_feats, lengths, jnp.ones((E, 1), jnp.float32),
         jnp.zeros((E, 16 - RB - 2), jnp.float32)], axis=1)

    au, b = _node_precompute(node_feats, W_scalar, W1a, W1b, W_up)

    nchunk = E // CH
    ei3 = edge_index.reshape(2, nchunk, CH).transpose(1, 0, 2)
    ri = edge_index[1].reshape(nchunk, CH)

    partials = []
    off_c = 0
    for (nc_s, eb_s), g_call, s_call in zip(STRIPES, _gathers, _scatters):
        ne_s = nc_s * CH
        e0 = off_c * CH
        gp, gr = g_call(au, b, ei3[off_c:off_c + nc_s])
        mji = _edge_mlp(gp, gr,
                        ef_aug[e0:e0 + ne_s],
                        edge_attrs[e0:e0 + ne_s],
                        W1c_aug, W2, b2[None, :], W3, eb_s)
        partials.append(s_call(mji, ri[off_c:off_c + nc_s]))
        off_c += nc_s

    out = _final(partials[0], W_out)
    return out.reshape(N, D, 1)


# speedup vs baseline: 1.0712x; 1.0046x over previous
import functools

import jax
import jax.numpy as jnp
from jax import lax
from jax.experimental import pallas as pl
from jax.experimental.pallas import tpu as pltpu
from jax.experimental.pallas import tpu_sc as plsc

N = 10000
E = 320000
D = 128
RB = 8
AVG_NEIGH = 32.0

NB = 2000

NC = 2
NS = 16
NW = NC * NS
CH = 128

STRIPES = ((2500, 8000),)

N_PAD = 10240
_RPT = N_PAD // NS
_ZR = 32

_sc_mesh = plsc.VectorSubcoreMesh(core_axis_name="c", subcore_axis_name="s")


def _node_kernel(nf_ref, wsc_ref, w1a_ref, w1b_ref, wup_ref, au_ref, b_ref):
    nf = nf_ref[...]
    ns = jnp.dot(nf, wsc_ref[...], preferred_element_type=jnp.float32)
    a = jnp.dot(ns, w1a_ref[...], preferred_element_type=jnp.float32)
    u = jnp.dot(nf, wup_ref[...], preferred_element_type=jnp.float32)
    ai = lax.bitcast_convert_type(a.astype(jnp.bfloat16).astype(jnp.float32),
                                  jnp.int32)
    ui = lax.bitcast_convert_type(u.astype(jnp.bfloat16).astype(jnp.float32),
                                  jnp.int32)
    au_ref[...] = lax.bitcast_convert_type(
        ai | lax.shift_right_logical(ui, 16), jnp.float32)
    b_ref[...] = jnp.dot(ns, w1b_ref[...], preferred_element_type=jnp.float32)


def _node_precompute(node_feats, W_scalar, W1a, W1b, W_up):
    grid = (N // NB,)
    blk = pl.BlockSpec((NB, D), lambda i: (i, 0))
    wblk = pl.BlockSpec((D, D), lambda i: (0, 0))
    return pl.pallas_call(
        _node_kernel,
        grid=grid,
        in_specs=[blk, wblk, wblk, wblk, wblk],
        out_specs=[blk, blk],
        out_shape=[jax.ShapeDtypeStruct((N, D), jnp.float32),
                   jax.ShapeDtypeStruct((N, D), jnp.float32)],
    )(node_feats, W_scalar, W1a, W1b, W_up)


def _make_gather(nchunk):
    base = nchunk // NW
    extra = nchunk - base * NW
    ne = nchunk * CH

    @functools.partial(
        pl.kernel,
        out_type=[jax.ShapeDtypeStruct((ne, D), jnp.float32),
                  jax.ShapeDtypeStruct((ne, D), jnp.float32)],
        mesh=_sc_mesh,
        scratch_types=[
            pltpu.VMEM((2, CH), jnp.int32),
            pltpu.VMEM((2, CH), jnp.int32),
            pltpu.VMEM((CH, D), jnp.float32),
            pltpu.VMEM((CH, D), jnp.float32),
            pltpu.VMEM((CH, D), jnp.float32),
            pltpu.VMEM((CH, D), jnp.float32),
            pltpu.SemaphoreType.DMA,
            pltpu.SemaphoreType.DMA,
            pltpu.SemaphoreType.DMA,
            pltpu.SemaphoreType.DMA,
            pltpu.SemaphoreType.DMA,
            pltpu.SemaphoreType.DMA,
        ],
    )
    def gather(au_hbm, b_hbm, ei_hbm, gp_hbm, gr_hbm,
               idx0, idx1, au0, au1, b0, b1, si0, si1, sg0, sg1, sw0, sw1):
        wid = lax.axis_index("s") * NC + lax.axis_index("c")

        def start_idx(j, idxb, sem):
            pltpu.async_copy(ei_hbm.at[wid + j * NW], idxb, sem)

        def wait_idx(idxb, sem):
            pltpu.make_async_copy(ei_hbm.at[0], idxb, sem).wait()

        def start_gather(idxb, aub, bb, sem):
            pltpu.async_copy(au_hbm.at[idxb.at[0]], aub, sem)
            pltpu.async_copy(b_hbm.at[idxb.at[1]], bb, sem)

        def wait_gather(idxb, aub, bb, sem):
            pltpu.make_async_copy(au_hbm.at[idxb.at[0]], aub, sem).wait()
            pltpu.make_async_copy(b_hbm.at[idxb.at[1]], bb, sem).wait()

        def start_write(j, aub, bb, sem):
            cid = wid + j * NW
            pltpu.async_copy(aub, gp_hbm.at[pl.ds(cid * CH, CH)], sem)
            pltpu.async_copy(bb, gr_hbm.at[pl.ds(cid * CH, CH)], sem)

        def wait_write(aub, bb, sem):
            pltpu.make_async_copy(aub, gp_hbm.at[pl.ds(0, CH)], sem).wait()
            pltpu.make_async_copy(bb, gr_hbm.at[pl.ds(0, CH)], sem).wait()

        start_idx(0, idx0, si0)
        start_idx(1, idx1, si1)

        def body(jj, carry):
            wait_idx(idx0, si0)

            @pl.when(jj > 0)
            def _():
                wait_write(au0, b0, sw0)

            start_gather(idx0, au0, b0, sg0)
            wait_idx(idx1, si1)

            @pl.when(jj > 0)
            def _():
                wait_write(au1, b1, sw1)

            start_gather(idx1, au1, b1, sg1)
            wait_gather(idx0, au0, b0, sg0)

            @pl.when(jj < base // 2 - 1)
            def _():
                start_idx(2 * jj + 2, idx0, si0)

            start_write(2 * jj, au0, b0, sw0)
            wait_gather(idx1, au1, b1, sg1)

            @pl.when(jj < base // 2 - 1)
            def _():
                start_idx(2 * jj + 3, idx1, si1)

            start_write(2 * jj + 1, au1, b1, sw1)
            return carry

        lax.fori_loop(0, base // 2, body, 0)
        wait_write(au0, b0, sw0)
        wait_write(au1, b1, sw1)

        if extra:
            @pl.when(wid < extra)
            def _():
                start_idx(base, idx0, si0)
                wait_idx(idx0, si0)
                start_gather(idx0, au0, b0, sg0)
                wait_gather(idx0, au0, b0, sg0)
                start_write(base, au0, b0, sw0)
                wait_write(au0, b0, sw0)

    return gather


def _make_scatter(nchunk):
    base = nchunk // NW
    extra = nchunk - base * NW

    @functools.partial(
        pl.kernel,
        out_type=jax.ShapeDtypeStruct((NC, N_PAD, D), jnp.float32),
        mesh=_sc_mesh,
        scratch_types=[
            pltpu.VMEM((CH,), jnp.int32),
            pltpu.VMEM((CH,), jnp.int32),
            pltpu.VMEM((CH, D), jnp.float32),
            pltpu.VMEM((CH, D), jnp.float32),
            pltpu.VMEM((_ZR, D), jnp.float32),
            pltpu.VMEM_SHARED((N_PAD, D), jnp.float32),
            pltpu.SemaphoreType.DMA,
            pltpu.SemaphoreType.DMA,
        ],
    )
    def scatter(mji_hbm, ri_hbm, out_hbm, idx0, idx1, rows0, rows1, zbuf,
                acc, sl0, sl1):
        c = lax.axis_index("c")
        s = lax.axis_index("s")
        wid = s * NC + c

        def zrow(i, carry):
            for k in range(D // 16):
                zbuf[i, pl.ds(k * 16, 16)] = jnp.zeros((16,), jnp.float32)
            return carry

        lax.fori_loop(0, _ZR, zrow, 0)
        for t in range(_RPT // _ZR):
            pltpu.sync_copy(zbuf, acc.at[pl.ds(s * _RPT + t * _ZR, _ZR)])
        plsc.subcore_barrier()

        def start_load(j, idxb, rowsb, sem):
            cid = wid + j * NW
            pltpu.async_copy(ri_hbm.at[cid], idxb, sem)
            pltpu.async_copy(mji_hbm.at[pl.ds(cid * CH, CH)], rowsb, sem)

        def wait_load(idxb, rowsb, sem):
            pltpu.make_async_copy(ri_hbm.at[0], idxb, sem).wait()
            pltpu.make_async_copy(mji_hbm.at[pl.ds(0, CH)], rowsb, sem).wait()

        start_load(0, idx0, rows0, sl0)
        start_load(1, idx1, rows1, sl1)

        def body(jj, carry):
            wait_load(idx0, rows0, sl0)
            pltpu.sync_copy(rows0, acc.at[idx0], add=True)

            @pl.when(jj < base // 2 - 1)
            def _():
                start_load(2 * jj + 2, idx0, rows0, sl0)

            wait_load(idx1, rows1, sl1)
            pltpu.sync_copy(rows1, acc.at[idx1], add=True)

            @pl.when(jj < base // 2 - 1)
            def _():
                start_load(2 * jj + 3, idx1, rows1, sl1)

            return carry

        lax.fori_loop(0, base // 2, body, 0)

        if extra:
            @pl.when(wid < extra)
            def _():
                start_load(base, idx0, rows0, sl0)
                wait_load(idx0, rows0, sl0)
                pltpu.sync_copy(rows0, acc.at[idx0], add=True)

        plsc.subcore_barrier()
        sl = pl.ds(s * _RPT, _RPT)
        pltpu.sync_copy(acc.at[sl], out_hbm.at[c, sl])

    return scatter


_gathers = tuple(_make_gather(nc) for nc, _ in STRIPES)
_scatters = tuple(_make_scatter(nc) for nc, _ in STRIPES)


def _mlp_kernel(gp_ref, gr_ref, ef_ref, ea_ref,
                w1c_ref, w2_ref, b2_ref, w3_ref, o_ref):
    xi = lax.bitcast_convert_type(gp_ref[...], jnp.int32)
    gs = lax.bitcast_convert_type(xi & jnp.int32(-65536), jnp.float32)
    u = lax.bitcast_convert_type(lax.shift_left(xi, 16), jnp.float32)
    pre = gs + gr_ref[...]
    pre = pre + jnp.dot(ef_ref[...], w1c_ref[...],
                        preferred_element_type=jnp.float32)
    h = pre * jax.nn.sigmoid(pre)
    pre2 = jnp.dot(h, w2_ref[...], preferred_element_type=jnp.float32) + b2_ref[...]
    h2 = pre2 * jax.nn.sigmoid(pre2)
    t = jnp.dot(h2, w3_ref[...], preferred_element_type=jnp.float32)
    o_ref[...] = u * ea_ref[...] * t


def _edge_mlp(gp, gr, ef_aug, ea, W1c_aug, W2, b2, W3, eb):
    ne = gp.shape[0]
    grid = (ne // eb,)
    eblk = pl.BlockSpec((eb, D), lambda i: (i, 0))
    wblk = pl.BlockSpec((D, D), lambda i: (0, 0))
    rblk = pl.BlockSpec((1, D), lambda i: (0, 0))
    return pl.pallas_call(
        _mlp_kernel,
        grid=grid,
        in_specs=[
            eblk, eblk,
            pl.BlockSpec((eb, RB + 8), lambda i: (i, 0)),
            pl.BlockSpec((eb, 1), lambda i: (i, 0)),
            pl.BlockSpec((RB + 8, D), lambda i: (0, 0)),
            wblk, rblk, wblk,
        ],
        out_specs=eblk,
        out_shape=jax.ShapeDtypeStruct((ne, D), jnp.float32),
    )(gp, gr, ef_aug, ea, W1c_aug, W2, b2, W3)


def _final_kernel(p0_ref, wout_ref, o_ref):
    m = p0_ref[0] + p0_ref[1]
    o_ref[...] = jnp.dot(m, wout_ref[...],
                         preferred_element_type=jnp.float32) * (1.0 / AVG_NEIGH)


def _final(p0, W_out):
    grid = (N // NB,)
    blk = pl.BlockSpec((NB, D), lambda i: (i, 0))
    pblk = pl.BlockSpec((NC, NB, D), lambda i: (0, i, 0))
    return pl.pallas_call(
        _final_kernel,
        grid=grid,
        in_specs=[pblk, pl.BlockSpec((D, D), lambda i: (0, 0))],
        out_specs=blk,
        out_shape=jax.ShapeDtypeStruct((N, D), jnp.float32),
    )(p0, W_out)


def kernel(node_feats, edge_attrs, edge_feats, lengths, W_scalar, W_up,
           W1, b1, W2, b2, W3, W_out, edge_index):
    W1a = W1[:D]
    W1b = W1[D:2 * D]
    W1c_aug = jnp.concatenate(
        [W1[2 * D:], b1[None, :], jnp.zeros((16 - RB - 2, D), jnp.float32)],
        axis=0)
    ef_aug = jnp.concatenate(
        [edge_feats, lengths, jnp.ones((E, 1), jnp.float32),
         jnp.zeros((E, 16 - RB - 2), jnp.float32)], axis=1)

    au, b = _node_precompute(node_feats, W_scalar, W1a, W1b, W_up)

    nchunk = E // CH
    ei3 = edge_index.reshape(2, nchunk, CH).transpose(1, 0, 2)
    ri = edge_index[1].reshape(nchunk, CH)

    partials = []
    off_c = 0
    for (nc_s, eb_s), g_call, s_call in zip(STRIPES, _gathers, _scatters):
        ne_s = nc_s * CH
        e0 = off_c * CH
        gp, gr = g_call(au, b, ei3[off_c:off_c + nc_s])
        mji = _edge_mlp(gp, gr,
                        ef_aug[e0:e0 + ne_s],
                        edge_attrs[e0:e0 + ne_s],
                        W1c_aug, W2, b2[None, :], W3, eb_s)
        partials.append(s_call(mji, ri[off_c:off_c + nc_s]))
        off_c += nc_s

    out = _final(partials[0], W_out)
    return out.reshape(N, D, 1)


# speedup vs baseline: 1.0725x; 1.0012x over previous
import functools

import jax
import jax.numpy as jnp
from jax import lax
from jax.experimental import pallas as pl
from jax.experimental.pallas import tpu as pltpu
from jax.experimental.pallas import tpu_sc as plsc

N = 10000
E = 320000
D = 128
RB = 8
AVG_NEIGH = 32.0

NB = 2000

NC = 2
NS = 16
NW = NC * NS
CH = 128

STRIPES = ((2500, 8000),)

N_PAD = 10240
_RPT = N_PAD // NS
_ZR = 32

_sc_mesh = plsc.VectorSubcoreMesh(core_axis_name="c", subcore_axis_name="s")


def _node_kernel(nf_ref, wsc_ref, w1a_ref, w1b_ref, wup_ref, au_ref, b_ref):
    nf = nf_ref[...]
    ns = jnp.dot(nf, wsc_ref[...], preferred_element_type=jnp.float32)
    a = jnp.dot(ns, w1a_ref[...], preferred_element_type=jnp.float32)
    u = jnp.dot(nf, wup_ref[...], preferred_element_type=jnp.float32)
    ai = lax.bitcast_convert_type(a.astype(jnp.bfloat16).astype(jnp.float32),
                                  jnp.int32)
    ui = lax.bitcast_convert_type(u.astype(jnp.bfloat16).astype(jnp.float32),
                                  jnp.int32)
    au_ref[...] = lax.bitcast_convert_type(
        ai | lax.shift_right_logical(ui, 16), jnp.float32)
    b_ref[...] = jnp.dot(ns, w1b_ref[...], preferred_element_type=jnp.float32)


def _node_precompute(node_feats, W_scalar, W1a, W1b, W_up):
    grid = (N // NB,)
    blk = pl.BlockSpec((NB, D), lambda i: (i, 0))
    wblk = pl.BlockSpec((D, D), lambda i: (0, 0))
    return pl.pallas_call(
        _node_kernel,
        grid=grid,
        in_specs=[blk, wblk, wblk, wblk, wblk],
        out_specs=[blk, blk],
        out_shape=[jax.ShapeDtypeStruct((N, D), jnp.float32),
                   jax.ShapeDtypeStruct((N, D), jnp.float32)],
    )(node_feats, W_scalar, W1a, W1b, W_up)


def _make_gather(nchunk):
    base = nchunk // NW
    extra = nchunk - base * NW
    ne = nchunk * CH

    nslot = 3
    assert base % nslot == 0

    @functools.partial(
        pl.kernel,
        out_type=[jax.ShapeDtypeStruct((ne, D), jnp.float32),
                  jax.ShapeDtypeStruct((ne, D), jnp.float32)],
        mesh=_sc_mesh,
        scratch_types=(
            [pltpu.VMEM((2, CH), jnp.int32)] * nslot
            + [pltpu.VMEM((CH, D), jnp.float32)] * (2 * nslot)
            + [pltpu.SemaphoreType.DMA] * (3 * nslot)
        ),
    )
    def gather(au_hbm, b_hbm, ei_hbm, gp_hbm, gr_hbm, *scr):
        idxs = scr[:nslot]
        aus = scr[nslot:2 * nslot]
        bs = scr[2 * nslot:3 * nslot]
        sis = scr[3 * nslot:4 * nslot]
        sgs = scr[4 * nslot:5 * nslot]
        sws = scr[5 * nslot:6 * nslot]
        wid = lax.axis_index("s") * NC + lax.axis_index("c")

        def start_idx(j, k):
            pltpu.async_copy(ei_hbm.at[wid + j * NW], idxs[k], sis[k])

        def wait_idx(k):
            pltpu.make_async_copy(ei_hbm.at[0], idxs[k], sis[k]).wait()

        def start_gather(k):
            pltpu.async_copy(au_hbm.at[idxs[k].at[0]], aus[k], sgs[k])
            pltpu.async_copy(b_hbm.at[idxs[k].at[1]], bs[k], sgs[k])

        def wait_gather(k):
            pltpu.make_async_copy(au_hbm.at[idxs[k].at[0]], aus[k], sgs[k]).wait()
            pltpu.make_async_copy(b_hbm.at[idxs[k].at[1]], bs[k], sgs[k]).wait()

        def start_write(j, k):
            cid = wid + j * NW
            pltpu.async_copy(aus[k], gp_hbm.at[pl.ds(cid * CH, CH)], sws[k])
            pltpu.async_copy(bs[k], gr_hbm.at[pl.ds(cid * CH, CH)], sws[k])

        def wait_write(k):
            pltpu.make_async_copy(aus[k], gp_hbm.at[pl.ds(0, CH)], sws[k]).wait()
            pltpu.make_async_copy(bs[k], gr_hbm.at[pl.ds(0, CH)], sws[k]).wait()

        for k in range(nslot):
            start_idx(k, k)

        def body(jj, carry):
            for k in range(nslot):
                wait_idx(k)

                @pl.when(jj > 0)
                def _(k=k):
                    wait_write(k)

                start_gather(k)
            for k in range(nslot):
                wait_gather(k)

                @pl.when(jj < base // nslot - 1)
                def _(jj=jj, k=k):
                    start_idx(nslot * jj + nslot + k, k)

                start_write(nslot * jj + k, k)
            return carry

        lax.fori_loop(0, base // nslot, body, 0)
        for k in range(nslot):
            wait_write(k)

        if extra:
            @pl.when(wid < extra)
            def _():
                start_idx(base, 0)
                wait_idx(0)
                start_gather(0)
                wait_gather(0)
                start_write(base, 0)
                wait_write(0)

    return gather


def _make_scatter(nchunk):
    base = nchunk // NW
    extra = nchunk - base * NW

    @functools.partial(
        pl.kernel,
        out_type=jax.ShapeDtypeStruct((NC, N_PAD, D), jnp.float32),
        mesh=_sc_mesh,
        scratch_types=[
            pltpu.VMEM((CH,), jnp.int32),
            pltpu.VMEM((CH,), jnp.int32),
            pltpu.VMEM((CH, D), jnp.float32),
            pltpu.VMEM((CH, D), jnp.float32),
            pltpu.VMEM((_ZR, D), jnp.float32),
            pltpu.VMEM_SHARED((N_PAD, D), jnp.float32),
            pltpu.SemaphoreType.DMA,
            pltpu.SemaphoreType.DMA,
        ],
    )
    def scatter(mji_hbm, ri_hbm, out_hbm, idx0, idx1, rows0, rows1, zbuf,
                acc, sl0, sl1):
        c = lax.axis_index("c")
        s = lax.axis_index("s")
        wid = s * NC + c

        def zrow(i, carry):
            for k in range(D // 16):
                zbuf[i, pl.ds(k * 16, 16)] = jnp.zeros((16,), jnp.float32)
            return carry

        lax.fori_loop(0, _ZR, zrow, 0)
        for t in range(_RPT // _ZR):
            pltpu.sync_copy(zbuf, acc.at[pl.ds(s * _RPT + t * _ZR, _ZR)])
        plsc.subcore_barrier()

        def start_load(j, idxb, rowsb, sem):
            cid = wid + j * NW
            pltpu.async_copy(ri_hbm.at[cid], idxb, sem)
            pltpu.async_copy(mji_hbm.at[pl.ds(cid * CH, CH)], rowsb, sem)

        def wait_load(idxb, rowsb, sem):
            pltpu.make_async_copy(ri_hbm.at[0], idxb, sem).wait()
            pltpu.make_async_copy(mji_hbm.at[pl.ds(0, CH)], rowsb, sem).wait()

        start_load(0, idx0, rows0, sl0)
        start_load(1, idx1, rows1, sl1)

        def body(jj, carry):
            wait_load(idx0, rows0, sl0)
            pltpu.sync_copy(rows0, acc.at[idx0], add=True)

            @pl.when(jj < base // 2 - 1)
            def _():
                start_load(2 * jj + 2, idx0, rows0, sl0)

            wait_load(idx1, rows1, sl1)
            pltpu.sync_copy(rows1, acc.at[idx1], add=True)

            @pl.when(jj < base // 2 - 1)
            def _():
                start_load(2 * jj + 3, idx1, rows1, sl1)

            return carry

        lax.fori_loop(0, base // 2, body, 0)

        if extra:
            @pl.when(wid < extra)
            def _():
                start_load(base, idx0, rows0, sl0)
                wait_load(idx0, rows0, sl0)
                pltpu.sync_copy(rows0, acc.at[idx0], add=True)

        plsc.subcore_barrier()
        sl = pl.ds(s * _RPT, _RPT)
        pltpu.sync_copy(acc.at[sl], out_hbm.at[c, sl])

    return scatter


_gathers = tuple(_make_gather(nc) for nc, _ in STRIPES)
_scatters = tuple(_make_scatter(nc) for nc, _ in STRIPES)


def _mlp_kernel(gp_ref, gr_ref, ef_ref, ea_ref,
                w1c_ref, w2_ref, b2_ref, w3_ref, o_ref):
    xi = lax.bitcast_convert_type(gp_ref[...], jnp.int32)
    gs = lax.bitcast_convert_type(xi & jnp.int32(-65536), jnp.float32)
    u = lax.bitcast_convert_type(lax.shift_left(xi, 16), jnp.float32)
    pre = gs + gr_ref[...]
    pre = pre + jnp.dot(ef_ref[...], w1c_ref[...],
                        preferred_element_type=jnp.float32)
    h = pre * jax.nn.sigmoid(pre)
    pre2 = jnp.dot(h, w2_ref[...], preferred_element_type=jnp.float32) + b2_ref[...]
    h2 = pre2 * jax.nn.sigmoid(pre2)
    t = jnp.dot(h2, w3_ref[...], preferred_element_type=jnp.float32)
    o_ref[...] = u * ea_ref[...] * t


def _edge_mlp(gp, gr, ef_aug, ea, W1c_aug, W2, b2, W3, eb):
    ne = gp.shape[0]
    grid = (ne // eb,)
    eblk = pl.BlockSpec((eb, D), lambda i: (i, 0))
    wblk = pl.BlockSpec((D, D), lambda i: (0, 0))
    rblk = pl.BlockSpec((1, D), lambda i: (0, 0))
    return pl.pallas_call(
        _mlp_kernel,
        grid=grid,
        in_specs=[
            eblk, eblk,
            pl.BlockSpec((eb, RB + 8), lambda i: (i, 0)),
            pl.BlockSpec((eb, 1), lambda i: (i, 0)),
            pl.BlockSpec((RB + 8, D), lambda i: (0, 0)),
            wblk, rblk, wblk,
        ],
        out_specs=eblk,
        out_shape=jax.ShapeDtypeStruct((ne, D), jnp.float32),
    )(gp, gr, ef_aug, ea, W1c_aug, W2, b2, W3)


def _final_kernel(p0_ref, wout_ref, o_ref):
    m = p0_ref[0] + p0_ref[1]
    o_ref[...] = jnp.dot(m, wout_ref[...],
                         preferred_element_type=jnp.float32) * (1.0 / AVG_NEIGH)


def _final(p0, W_out):
    grid = (N // NB,)
    blk = pl.BlockSpec((NB, D), lambda i: (i, 0))
    pblk = pl.BlockSpec((NC, NB, D), lambda i: (0, i, 0))
    return pl.pallas_call(
        _final_kernel,
        grid=grid,
        in_specs=[pblk, pl.BlockSpec((D, D), lambda i: (0, 0))],
        out_specs=blk,
        out_shape=jax.ShapeDtypeStruct((N, D), jnp.float32),
    )(p0, W_out)


def kernel(node_feats, edge_attrs, edge_feats, lengths, W_scalar, W_up,
           W1, b1, W2, b2, W3, W_out, edge_index):
    W1a = W1[:D]
    W1b = W1[D:2 * D]
    W1c_aug = jnp.concatenate(
        [W1[2 * D:], b1[None, :], jnp.zeros((16 - RB - 2, D), jnp.float32)],
        axis=0)
    ef_aug = jnp.concatenate(
        [edge_feats, lengths, jnp.ones((E, 1), jnp.float32),
         jnp.zeros((E, 16 - RB - 2), jnp.float32)], axis=1)

    au, b = _node_precompute(node_feats, W_scalar, W1a, W1b, W_up)

    nchunk = E // CH
    ei3 = edge_index.reshape(2, nchunk, CH).transpose(1, 0, 2)
    ri = edge_index[1].reshape(nchunk, CH)

    partials = []
    off_c = 0
    for (nc_s, eb_s), g_call, s_call in zip(STRIPES, _gathers, _scatters):
        ne_s = nc_s * CH
        e0 = off_c * CH
        gp, gr = g_call(au, b, ei3[off_c:off_c + nc_s])
        mji = _edge_mlp(gp, gr,
                        ef_aug[e0:e0 + ne_s],
                        edge_attrs[e0:e0 + ne_s],
                        W1c_aug, W2, b2[None, :], W3, eb_s)
        partials.append(s_call(mji, ri[off_c:off_c + nc_s]))
        off_c += nc_s

    out = _final(partials[0], W_out)
    return out.reshape(N, D, 1)
